# Initial kernel scaffold; baseline (speedup 1.0000x reference)
import jax
import jax.numpy as jnp
from jax.experimental import pallas as pl


def kernel(x, edge_index, W1, a_src1, a_dst1, b1, W2, a_src2, a_dst2, b2, W3, a_src3, a_dst3, b3):
    raise NotImplementedError("write your pallas kernel here")


# speedup vs baseline: 32.9065x; 32.9065x over previous
import functools

import jax
import jax.numpy as jnp
from jax import lax
from jax.experimental import pallas as pl
from jax.experimental.pallas import tpu as pltpu
from jax.experimental.pallas import tpu_sc as plsc

N = 10000
E = 320000
D = 128
HID = 128
NCLS = 4

NC = 2
NS = 16
NW = NC * NS
K = 128
NCH = 81
EP = NW * NCH * K
NP = 10112
DUMMY = N
RPT = NP // NS
BR = NP // 8
CW1 = HID + 16
CW3 = 16


def _make_sc_layer(cw):
    mesh = plsc.VectorSubcoreMesh(core_axis_name="c", subcore_axis_name="s")

    @functools.partial(
        pl.kernel,
        mesh=mesh,
        compiler_params=pltpu.CompilerParams(use_tc_tiling_on_sc=False),
        out_type=jax.ShapeDtypeStruct((NC, NP, cw), jnp.float32),
        scratch_types=[
            pltpu.VMEM((NCH, K), jnp.int32),
            pltpu.VMEM((NCH, K), jnp.int32),
            pltpu.VMEM((K,), jnp.float32),
            pltpu.VMEM((K,), jnp.float32),
            pltpu.VMEM((K,), jnp.float32),
            pltpu.VMEM((K, cw), jnp.float32),
            pltpu.VMEM_SHARED((NP, cw), jnp.float32),
            pltpu.SemaphoreType.DMA,
            pltpu.SemaphoreType.DMA,
            pltpu.SemaphoreType.DMA,
        ],
    )
    def sc_layer(src_hbm, dst_hbm, asrc_hbm, adst_hbm, h_hbm, out_hbm,
                 sidx_v, didx_v, avbuf, dvbuf, pbuf, rows_v, out_sh,
                 sem, sem_a, sem_d):
        c = lax.axis_index("c")
        s = lax.axis_index("s")
        wid = c * NS + s

        pltpu.sync_copy(src_hbm.at[wid], sidx_v)
        pltpu.sync_copy(dst_hbm.at[wid], didx_v)

        def zrow(r, carry):
            for j in range(cw // 16):
                rows_v[r, pl.ds(j * 16, 16)] = jnp.zeros((16,), jnp.float32)
            return carry
        lax.fori_loop(0, K, zrow, 0)
        row0 = s * RPT
        nfull = RPT // K
        rem = RPT - nfull * K
        for t in range(nfull):
            pltpu.sync_copy(rows_v, out_sh.at[pl.ds(row0 + t * K, K)])
        pltpu.sync_copy(rows_v.at[pl.ds(0, rem)],
                        out_sh.at[pl.ds(row0 + nfull * K, rem)])
        plsc.subcore_barrier()

        def chunk(ch, carry):
            cp_rows = pltpu.async_copy(h_hbm.at[sidx_v.at[ch]], rows_v, sem)
            cp_a = pltpu.async_copy(asrc_hbm.at[sidx_v.at[ch]], avbuf, sem_a)
            cp_d = pltpu.async_copy(adst_hbm.at[didx_v.at[ch]], dvbuf, sem_d)
            cp_a.wait()
            cp_d.wait()
            for g in range(K // 16):
                lanes = pl.ds(g * 16, 16)
                e = avbuf[lanes] + dvbuf[lanes]
                e = jnp.where(e < 0.0, e * 0.2, e)
                pbuf[lanes] = jnp.exp(e)
            cp_rows.wait()
            def srow(g2, c2):
                pv = pbuf[pl.ds(g2 * 16, 16)]
                for rl in range(16):
                    pr = jnp.broadcast_to(pv[rl], (16,))
                    r = g2 * 16 + rl
                    for j in range(cw // 16):
                        sl = pl.ds(j * 16, 16)
                        rows_v[r, sl] = rows_v[r, sl] * pr
                return c2
            lax.fori_loop(0, K // 16, srow, 0)
            pltpu.sync_copy(rows_v, out_sh.at[didx_v.at[ch]], add=True)
            return carry
        lax.fori_loop(0, NCH, chunk, 0)

        plsc.subcore_barrier()
        pltpu.sync_copy(out_sh.at[pl.ds(row0, RPT)],
                        out_hbm.at[c, pl.ds(row0, RPT)])

    return sc_layer


_sc_layer_wide = _make_sc_layer(CW1)
_sc_layer_narrow = _make_sc_layer(CW3)


def _tail(h, asv, adv):
    lane = lax.broadcasted_iota(jnp.int32, (BR, 16), 1)
    z = jnp.zeros((BR, 16), jnp.float32)
    t = jnp.where(lane == 0, 1.0, z)
    t = jnp.where(lane == 1, asv[:, None], t)
    return jnp.where(lane == 2, adv[:, None], t)


def _tc0_body(x_ref, w_ref, as_ref, ad_ref, h_ref, asrc_ref, adst_ref):
    h = jnp.dot(x_ref[...], w_ref[...], preferred_element_type=jnp.float32)
    asv = jnp.dot(h, as_ref[0])
    adv = jnp.dot(h, ad_ref[0])
    h_ref[:, 0:HID] = h
    h_ref[:, HID:HID + 16] = _tail(h, asv, adv)
    asrc_ref[...] = asv[None, None, :]
    adst_ref[...] = adv[None, None, :]


def _tc_mid_body(parts_ref, b_ref, w_ref, as_ref, ad_ref,
                 h_ref, asrc_ref, adst_ref):
    sacc = parts_ref[0] + parts_ref[1]
    h_in = sacc[:, 0:HID] / (sacc[:, HID:HID + 1] + 1e-16) + b_ref[...]
    h_in = jnp.where(h_in > 0.0, h_in, jnp.exp(jnp.minimum(h_in, 0.0)) - 1.0)
    h = jnp.dot(h_in, w_ref[...], preferred_element_type=jnp.float32)
    asv = jnp.dot(h, as_ref[0])
    adv = jnp.dot(h, ad_ref[0])
    h_ref[:, 0:HID] = h
    h_ref[:, HID:HID + 16] = _tail(h, asv, adv)
    asrc_ref[...] = asv[None, None, :]
    adst_ref[...] = adv[None, None, :]


def _tc2_body(parts_ref, b_ref, w_ref, as_ref, ad_ref,
              h_ref, asrc_ref, adst_ref):
    sacc = parts_ref[0] + parts_ref[1]
    h_in = sacc[:, 0:HID] / (sacc[:, HID:HID + 1] + 1e-16) + b_ref[...]
    h_in = jnp.where(h_in > 0.0, h_in, jnp.exp(jnp.minimum(h_in, 0.0)) - 1.0)
    g = jnp.dot(h_in, w_ref[...], preferred_element_type=jnp.float32)
    asv = jnp.dot(g, as_ref[0])
    adv = jnp.dot(g, ad_ref[0])
    lane = lax.broadcasted_iota(jnp.int32, (BR, CW3), 1)
    h_ref[...] = jnp.where(lane == CW3 - 1, 1.0, g)
    asrc_ref[...] = asv[None, None, :]
    adst_ref[...] = adv[None, None, :]


def _tc3_body(parts_ref, b_ref, out_ref):
    sacc = parts_ref[0] + parts_ref[1]
    g = sacc[:, 0:CW3] / (sacc[:, CW3 - 1:CW3] + 1e-16) + b_ref[...]
    lane = lax.broadcasted_iota(jnp.int32, (BR, CW3), 1)
    gm = jnp.where(lane < NCLS, g, -jnp.inf)
    m = jnp.max(gm, axis=1, keepdims=True)
    ex = jnp.exp(gm - m)
    lse = jnp.log(jnp.sum(ex, axis=1, keepdims=True))
    out_ref[...] = gm - m - lse


def _row_blocked(width):
    return pl.BlockSpec((BR, width), lambda i: (i, 0))


def _full(shape):
    return pl.BlockSpec(shape, lambda i: tuple(0 for _ in shape))


_tc0 = pl.pallas_call(
    _tc0_body,
    grid=(NP // BR,),
    in_specs=[_row_blocked(D), _full((D, HID)), _full((1, HID)), _full((1, HID))],
    out_specs=[_row_blocked(CW1),
               pl.BlockSpec((1, 1, BR), lambda i: (i, 0, 0)),
               pl.BlockSpec((1, 1, BR), lambda i: (i, 0, 0))],
    out_shape=[jax.ShapeDtypeStruct((NP, CW1), jnp.float32),
               jax.ShapeDtypeStruct((NP // BR, 1, BR), jnp.float32),
               jax.ShapeDtypeStruct((NP // BR, 1, BR), jnp.float32)],
)

_tc_mid = pl.pallas_call(
    _tc_mid_body,
    grid=(NP // BR,),
    in_specs=[pl.BlockSpec((2, BR, CW1), lambda i: (0, i, 0)),
              _full((1, HID)), _full((HID, HID)), _full((1, HID)), _full((1, HID))],
    out_specs=[_row_blocked(CW1),
               pl.BlockSpec((1, 1, BR), lambda i: (i, 0, 0)),
               pl.BlockSpec((1, 1, BR), lambda i: (i, 0, 0))],
    out_shape=[jax.ShapeDtypeStruct((NP, CW1), jnp.float32),
               jax.ShapeDtypeStruct((NP // BR, 1, BR), jnp.float32),
               jax.ShapeDtypeStruct((NP // BR, 1, BR), jnp.float32)],
)

_tc2 = pl.pallas_call(
    _tc2_body,
    grid=(NP // BR,),
    in_specs=[pl.BlockSpec((2, BR, CW1), lambda i: (0, i, 0)),
              _full((1, HID)), _full((HID, CW3)), _full((1, CW3)), _full((1, CW3))],
    out_specs=[_row_blocked(CW3),
               pl.BlockSpec((1, 1, BR), lambda i: (i, 0, 0)),
               pl.BlockSpec((1, 1, BR), lambda i: (i, 0, 0))],
    out_shape=[jax.ShapeDtypeStruct((NP, CW3), jnp.float32),
               jax.ShapeDtypeStruct((NP // BR, 1, BR), jnp.float32),
               jax.ShapeDtypeStruct((NP // BR, 1, BR), jnp.float32)],
)

_tc3 = pl.pallas_call(
    _tc3_body,
    grid=(NP // BR,),
    in_specs=[pl.BlockSpec((2, BR, CW3), lambda i: (0, i, 0)),
              _full((1, CW3))],
    out_specs=_row_blocked(CW3),
    out_shape=jax.ShapeDtypeStruct((NP, CW3), jnp.float32),
)


def kernel(x, edge_index, W1, a_src1, a_dst1, b1,
           W2, a_src2, a_dst2, b2, W3, a_src3, a_dst3, b3):
    loops = jnp.arange(N, dtype=jnp.int32)
    pad = jnp.full((EP - E - N,), DUMMY, jnp.int32)
    src = jnp.concatenate([edge_index[0].astype(jnp.int32), loops, pad])
    dst = jnp.concatenate([edge_index[1].astype(jnp.int32), loops, pad])
    src = src.reshape(NW, NCH, K)
    dst = dst.reshape(NW, NCH, K)

    x_pad = jnp.pad(x, ((0, NP - N), (0, 0)))
    w3p = jnp.pad(W3, ((0, 0), (0, CW3 - NCLS)))
    a3sp = jnp.pad(a_src3, (0, CW3 - NCLS))[None, :]
    a3dp = jnp.pad(a_dst3, (0, CW3 - NCLS))[None, :]
    b3p = jnp.pad(b3, (0, CW3 - NCLS))[None, :]

    h1, as1, ad1 = _tc0(x_pad, W1, a_src1[None, :], a_dst1[None, :])
    parts1 = _sc_layer_wide(src, dst, as1.reshape(NP), ad1.reshape(NP), h1)
    h2, as2, ad2 = _tc_mid(parts1, b1[None, :], W2,
                           a_src2[None, :], a_dst2[None, :])
    parts2 = _sc_layer_wide(src, dst, as2.reshape(NP), ad2.reshape(NP), h2)
    h3, as3, ad3 = _tc2(parts2, b2[None, :], w3p, a3sp, a3dp)
    parts3 = _sc_layer_narrow(src, dst, as3.reshape(NP), ad3.reshape(NP), h3)
    out = _tc3(parts3, b3p)
    return out[:N, :NCLS]
